# Initial kernel scaffold; baseline (speedup 1.0000x reference)
import jax
import jax.numpy as jnp
from jax.experimental import pallas as pl


def kernel(x, pos, edge_index, batch, cluster0, cluster1, params):
    raise NotImplementedError("write your pallas kernel here")


# speedup vs baseline: 1.7634x; 1.7634x over previous
import functools

import jax
import jax.numpy as jnp
from jax import lax
from jax.experimental import pallas as pl
from jax.experimental.pallas import tpu as pltpu
from jax.experimental.pallas import tpu_sc as plsc

N0, N1, N2 = 10000, 2500, 625
E = 160000
H = 128
A = 4
AH = A * H

NC, NS = 2, 16
NW = NC * NS

N1P = 2560
N2P = 640

EP = 163840
MGN = 32768

Y0 = 0.28209479177387814
C1 = 0.48860251190291987

f32 = jnp.float32


@functools.lru_cache(maxsize=None)
def _gather_fn(V, D, M):
    assert M % (NW * 1024) == 0 and D == 128
    per = M // NW
    mesh = plsc.VectorSubcoreMesh(core_axis_name="c", subcore_axis_name="s")

    def body(tab, idx2, out, ib, rows_v, sem):
        wid = lax.axis_index("s") * NC + lax.axis_index("c")
        base = pl.multiple_of(wid * per, 1024)
        rbase = pl.multiple_of(wid * (per // 128), 8)
        for g in range(per // 1024):
            pltpu.sync_copy(idx2.at[pl.ds(rbase + g * 8, 8)], ib)
            for h in range(2):
                cps = [
                    pltpu.async_copy(
                        tab.at[ib.at[h * 4 + j]],
                        rows_v.at[pl.ds(j * 128, 128)], sem)
                    for j in range(4)
                ]
                for cp in cps:
                    cp.wait()
                pltpu.sync_copy(
                    rows_v,
                    out.at[pl.ds(base + g * 1024 + h * 512, 512)])

    return pl.kernel(
        body,
        out_type=jax.ShapeDtypeStruct((M, D), f32),
        mesh=mesh,
        scratch_types=[
            pltpu.VMEM((8, 128), jnp.int32),
            pltpu.VMEM((512, D), f32),
            pltpu.SemaphoreType.DMA,
        ],
    )


def _gather(table, idx):
    V, D = table.shape
    (M,) = idx.shape
    return _gather_fn(V, D, M)(table, idx.reshape(M // 128, 128))


@functools.lru_cache(maxsize=None)
def _scatter_fn(M, NPs):
    D = 128
    assert M % (NW * 1024) == 0 and NPs % 128 == 0
    per = M // NW
    pr = NPs // NS
    mesh = plsc.VectorSubcoreMesh(core_axis_name="c", subcore_axis_name="s")

    def body(vals, idx2, zeros, out, acc, ib, rows_v, sem):
        cid = lax.axis_index("c")
        sid = lax.axis_index("s")
        row0 = pl.multiple_of(sid * pr, 8)
        pltpu.sync_copy(zeros.at[pl.ds(row0, pr)],
                        acc.at[pl.ds(row0, pr)])
        plsc.subcore_barrier()
        base = pl.multiple_of(cid * (M // 2) + sid * per, 1024)
        rbase = pl.multiple_of(cid * (M // 256) + sid * (per // 128), 8)
        for g in range(per // 1024):
            pltpu.sync_copy(idx2.at[pl.ds(rbase + g * 8, 8)], ib)
            for j in range(8):
                rv = rows_v.at[pl.ds(0, 128)]
                pltpu.sync_copy(
                    vals.at[pl.ds(base + g * 1024 + j * 128, 128)], rv)
                pltpu.sync_copy(rv, acc.at[ib.at[j]], add=True)
        plsc.subcore_barrier()
        pltpu.sync_copy(acc.at[pl.ds(row0, pr)],
                        out.at[cid, pl.ds(row0, pr)])

    return pl.kernel(
        body,
        out_type=jax.ShapeDtypeStruct((2, NPs, D), f32),
        mesh=mesh,
        scratch_types=[
            pltpu.VMEM_SHARED((NPs, D), f32),
            pltpu.VMEM((8, 128), jnp.int32),
            pltpu.VMEM((128, D), f32),
            pltpu.SemaphoreType.DMA,
        ],
    )


def _scatter_add(vals, idx, NP):
    M, D = vals.shape
    assert D == 128
    NPs = (NP + 127) // 128 * 128
    out = _scatter_fn(M, NPs)(vals, idx.reshape(M // 128, 128),
                              jnp.zeros((NPs, D), f32))
    return out[0][:NP], out[1][:NP]


@functools.lru_cache(maxsize=None)
def _compose_fn(V, M):
    assert M % (NW * 1024) == 0
    per = M // NW
    mesh = plsc.VectorSubcoreMesh(core_axis_name="c", subcore_axis_name="s")

    def body(tab, idx2, out, ib, gat_v, sem):
        wid = lax.axis_index("s") * NC + lax.axis_index("c")
        base = pl.multiple_of(wid * per, 1024)
        rbase = pl.multiple_of(wid * (per // 128), 8)
        for g in range(per // 1024):
            pltpu.sync_copy(idx2.at[pl.ds(rbase + g * 8, 8)], ib)
            cps = [
                pltpu.async_copy(tab.at[ib.at[j]],
                                 gat_v.at[pl.ds(j * 128, 128)], sem)
                for j in range(8)
            ]
            for cp in cps:
                cp.wait()
            pltpu.sync_copy(gat_v, out.at[pl.ds(base + g * 1024, 1024)])

    return pl.kernel(
        body,
        out_type=jax.ShapeDtypeStruct((M,), jnp.int32),
        mesh=mesh,
        scratch_types=[
            pltpu.VMEM((8, 128), jnp.int32),
            pltpu.VMEM((1024,), jnp.int32),
            pltpu.SemaphoreType.DMA,
        ],
    )


def _compose(table, idx):
    (V,) = table.shape
    (M,) = idx.shape
    return _compose_fn(V, M)(table, idx.reshape(M // 128, 128))


def _swish(v):
    return v * jax.nn.sigmoid(v)


@functools.lru_cache(maxsize=None)
def _geom_fn(M, B):
    nb = M // B

    def body(ps_ref, pd_ref, out_ref, pad_ref):
        d = ps_ref[:, 0:3] - pd_ref[:, 0:3]
        l2 = jnp.sum(d * d, axis=1, keepdims=True)
        l = jnp.sqrt(l2)
        n = d * (C1 / (l + 1e-8))
        one = jnp.ones((B, 1), f32)
        g = jnp.concatenate(
            [Y0 * one, n, l, one, jnp.zeros((B, 10), f32)], axis=1)
        if M > E:
            i = pl.program_id(0)
            rows = i * B + lax.broadcasted_iota(jnp.int32, (B, 1), 0)
            g = jnp.where(rows < E, g, 0.0)
        out_ref[...] = g
        pad_ref[...] = jnp.concatenate([g, jnp.zeros((B, 112), f32)], axis=1)

    return pl.pallas_call(
        body,
        grid=(nb,),
        in_specs=[pl.BlockSpec((B, 128), lambda i: (i, 0))] * 2,
        out_specs=[pl.BlockSpec((B, 16), lambda i: (i, 0)),
                   pl.BlockSpec((B, 128), lambda i: (i, 0))],
        out_shape=[jax.ShapeDtypeStruct((M, 16), f32),
                   jax.ShapeDtypeStruct((M, 128), f32)],
    )


@functools.lru_cache(maxsize=None)
def _edge_fn(nparts, M, B):
    nb = M // B

    def body(*refs):
        geom_ref = refs[0]
        part_refs = refs[1:1 + nparts]
        w1_refs = refs[1 + nparts:1 + 2 * nparts]
        w1len_ref = refs[1 + 2 * nparts]
        w2_ref = refs[2 + 2 * nparts]
        out_ref = refs[3 + 2 * nparts]

        g = geom_ref[...]
        l = g[:, 4:5]
        P = l * w1len_ref[0:1, :]
        for p_ref, w_ref in zip(part_refs, w1_refs):
            P = P + jnp.dot(p_ref[...], w_ref[...],
                            preferred_element_type=f32)
        m1 = jnp.zeros((B, H), f32)
        for a in range(A):
            m1 = m1 + g[:, a:a + 1] * P[:, a * H:(a + 1) * H]
        m1 = _swish(m1)
        P2 = jnp.dot(m1, w2_ref[...], preferred_element_type=f32)
        m2 = jnp.zeros((B, H), f32)
        for a in range(A):
            m2 = m2 + g[:, a:a + 1] * P2[:, a * H:(a + 1) * H]
        m2 = _swish(m2)
        if M > E:
            i = pl.program_id(0)
            rows = i * B + lax.broadcasted_iota(jnp.int32, (B, 1), 0)
            m2 = jnp.where(rows < E, m2, 0.0)
        out_ref[...] = m2

    in_specs = (
        [pl.BlockSpec((B, 16), lambda i: (i, 0))]
        + [pl.BlockSpec((B, H), lambda i: (i, 0))] * nparts
        + [pl.BlockSpec((H, AH), lambda i: (0, 0))] * nparts
        + [pl.BlockSpec((8, AH), lambda i: (0, 0))]
        + [pl.BlockSpec((H, AH), lambda i: (0, 0))]
    )
    return pl.pallas_call(
        body,
        grid=(nb,),
        in_specs=in_specs,
        out_specs=pl.BlockSpec((B, H), lambda i: (i, 0)),
        out_shape=jax.ShapeDtypeStruct((M, H), f32),
    )


@functools.lru_cache(maxsize=None)
def _update_fn(nparts, NP, B, n_real, residual, stats, final, pair=False):
    nb = NP // B
    nw_extra = 2 if final else 0
    nw = nparts - 1 if pair else nparts

    def body(*refs):
        part_refs = refs[:nparts]
        attr_ref = refs[nparts]
        w_refs = refs[nparts + 1:nparts + nw + 1]
        idx = nparts + nw + 1
        if final:
            wamb1_ref, wamb2_ref = refs[idx], refs[idx + 1]
            idx += 2
        out_ref = refs[idx]
        g = attr_ref[...]

        def tp(v, w_ref):
            Pv = jnp.dot(v, w_ref[...], preferred_element_type=f32)
            r = jnp.zeros((B, H), f32)
            for a in range(A):
                r = r + g[:, a:a + 1] * Pv[:, a * H:(a + 1) * H]
            return r

        P = jnp.zeros((B, AH), f32)
        if pair:
            vals = [r[...] for r in part_refs[:nw - 1]]
            vals.append(part_refs[nw - 1][...] + part_refs[nw][...])
        else:
            vals = [r[...] for r in part_refs]
        for v, w_ref in zip(vals, w_refs):
            P = P + jnp.dot(v, w_ref[...], preferred_element_type=f32)
        y = jnp.zeros((B, H), f32)
        for a in range(A):
            y = y + g[:, a:a + 1] * P[:, a * H:(a + 1) * H]
        if residual:
            y = y + part_refs[0][...]
        if final:
            y = _swish(tp(y, wamb1_ref))
            y = tp(y, wamb2_ref)
        if n_real < NP:
            i = pl.program_id(0)
            rows = i * B + lax.broadcasted_iota(jnp.int32, (B, 1), 0)
            y = jnp.where(rows < n_real, y, 0.0)
        out_ref[...] = y
        if stats:
            refs[idx + 1][...] = jnp.sum(y, axis=0)[None, None, :]
            refs[idx + 2][...] = jnp.sum(y * y, axis=0)[None, None, :]

    in_specs = (
        [pl.BlockSpec((B, H), lambda i: (i, 0))] * nparts
        + [pl.BlockSpec((B, 16), lambda i: (i, 0))]
        + [pl.BlockSpec((H, AH), lambda i: (0, 0))] * (nw + nw_extra)
    )
    out_shape = [jax.ShapeDtypeStruct((NP, H), f32)]
    out_specs = [pl.BlockSpec((B, H), lambda i: (i, 0))]
    if stats:
        out_shape += [jax.ShapeDtypeStruct((nb, 1, H), f32)] * 2
        out_specs += [pl.BlockSpec((1, 1, H), lambda i: (i, 0, 0))] * 2
    return pl.pallas_call(
        body,
        grid=(nb,),
        in_specs=in_specs,
        out_specs=out_specs,
        out_shape=out_shape,
    )


@functools.lru_cache(maxsize=None)
def _norm_fn(NP, B, n_real):
    nb = NP // B

    def body(y_ref, s_ref, q_ref, out_ref):
        s = jnp.sum(s_ref[...], axis=(0, 1))
        q = jnp.sum(q_ref[...], axis=(0, 1))
        mu = s / n_real
        var = jnp.maximum(q / n_real - mu * mu, 0.0)
        sd = jnp.sqrt(var) + 1e-5
        out = (y_ref[...] - mu[None, :]) / sd[None, :]
        if n_real < NP:
            i = pl.program_id(0)
            rows = i * B + lax.broadcasted_iota(jnp.int32, (B, 1), 0)
            out = jnp.where(rows < n_real, out, 0.0)
        out_ref[...] = out

    return pl.pallas_call(
        body,
        grid=(nb,),
        in_specs=[
            pl.BlockSpec((B, H), lambda i: (i, 0)),
            pl.BlockSpec((nb, 1, H), lambda i: (0, 0, 0)),
            pl.BlockSpec((nb, 1, H), lambda i: (0, 0, 0)),
        ],
        out_specs=pl.BlockSpec((B, H), lambda i: (i, 0)),
        out_shape=jax.ShapeDtypeStruct((NP, H), f32),
    )


@functools.lru_cache(maxsize=None)
def _attr_fin_fn(NP, B):
    nb = NP // B

    def body(s0_ref, s1_ref, out_ref):
        s = s0_ref[...] + s1_ref[...]
        cnt = jnp.maximum(s[:, 5:6], 1.0)
        out_ref[...] = jnp.concatenate(
            [s[:, 0:4] / cnt, jnp.zeros((B, 12), f32)], axis=1)

    return pl.pallas_call(
        body,
        grid=(nb,),
        in_specs=[pl.BlockSpec((B, 128), lambda i: (i, 0))] * 2,
        out_specs=pl.BlockSpec((B, 16), lambda i: (i, 0)),
        out_shape=jax.ShapeDtypeStruct((NP, 16), f32),
    )


@functools.lru_cache(maxsize=None)
def _pool_fin_fn(NP, B, n_real):
    nb = NP // B

    def body(xs0_ref, xs1_ref, ps0_ref, ps1_ref, xc_ref, pc_ref):
        ps = ps0_ref[...] + ps1_ref[...]
        cnt = jnp.maximum(ps[:, 3:4], 1.0)
        xc_ref[...] = (xs0_ref[...] + xs1_ref[...]) / cnt
        pc = ps / cnt
        i = pl.program_id(0)
        rows = i * B + lax.broadcasted_iota(jnp.int32, (B, 1), 0)
        one = jnp.where(rows < n_real, 1.0, 0.0)
        pc_ref[...] = jnp.concatenate([pc[:, 0:3], one, pc[:, 4:]], axis=1)

    return pl.pallas_call(
        body,
        grid=(nb,),
        in_specs=[pl.BlockSpec((B, H), lambda i: (i, 0))] * 2
        + [pl.BlockSpec((B, 128), lambda i: (i, 0))] * 2,
        out_specs=[
            pl.BlockSpec((B, H), lambda i: (i, 0)),
            pl.BlockSpec((B, 128), lambda i: (i, 0)),
        ],
        out_shape=[
            jax.ShapeDtypeStruct((NP, H), f32),
            jax.ShapeDtypeStruct((NP, 128), f32),
        ],
    )


def _wprep(W, part_rows):
    d_in = W.shape[0] // A
    Wc = W.reshape(d_in, A, H).reshape(d_in, AH)
    parts = [Wc[r:r + H] for r in part_rows]
    wlen = None
    if d_in % H == 1:
        wlen = jnp.pad(Wc[d_in - 1:d_in], ((0, 7), (0, 0)))
    return parts, wlen


def _edge_block(m):
    return 1000 if m % 1000 == 0 else 640


def _layer(src_gath, dst_gath, node_parts, geom, nattr, dst_idx,
           NP, n_real, Wm1, Wm2, Wu, residual, norm, BN,
           final_ws=None):
    parts = src_gath + dst_gath
    np_ = len(parts)
    w1p, w1len = _wprep(Wm1, [i * H for i in range(np_)])
    w2p, _ = _wprep(Wm2, [0])
    m = _edge_fn(np_, EP, 1024)(geom, *parts, *w1p, w1len, w2p[0])
    agg_a, agg_b = _scatter_add(m, dst_idx, NP)
    nparts = node_parts + [agg_a, agg_b]
    wup, _ = _wprep(Wu, [i * H for i in range(len(node_parts) + 1)])
    stats = norm
    args = nparts + [nattr] + wup
    if final_ws is not None:
        a1, _ = _wprep(final_ws[0], [0])
        a2, _ = _wprep(final_ws[1], [0])
        args += [a1[0], a2[0]]
    res = _update_fn(len(nparts), NP, BN, n_real, residual,
                     stats, final_ws is not None, pair=True)(*args)
    if stats:
        y, s, q = res
        return _norm_fn(NP, BN, n_real)(y, s, q)
    return res[0]


def _o3(pos_t, src, dst, NP, BN):
    ps = _gather(pos_t, src)
    pd = _gather(pos_t, dst)
    geom, gpad = _geom_fn(EP, 1024)(ps, pd)
    asum = _scatter_add(gpad, dst, NP)
    nattr = _attr_fin_fn(NP, BN)(*asum)
    return geom, nattr


def _pool(x_vals, pos_vals, cl_idx, NPc, BNc, n_real_c):
    m = x_vals.shape[0]
    x_vals = jnp.pad(x_vals, ((0, MGN - m), (0, 0)))
    pos_vals = jnp.pad(pos_vals, ((0, MGN - m), (0, 0)))
    cl_idx = jnp.pad(cl_idx, (0, MGN - cl_idx.shape[0]))
    xs = _scatter_add(x_vals, cl_idx, NPc)
    ps = _scatter_add(pos_vals, cl_idx, NPc)
    return _pool_fin_fn(NPc, BNc, n_real_c)(*xs, *ps)


def kernel(x, pos, edge_index, batch, cluster0, cluster1, params):
    del batch
    p = params
    eip = jnp.pad(edge_index, ((0, 0), (0, EP - E)))
    src0 = eip[0]
    dst0 = eip[1]

    pos0 = jnp.concatenate(
        [pos, jnp.ones((N0, 1), f32), jnp.zeros((N0, 124), f32)], axis=1)

    geom0, nattr0 = _o3(pos0, src0, dst0, N0, 1000)
    we, _ = _wprep(p['W_emb'], [0])
    x0 = _update_fn(1, N0, 1000, N0, False, False, False)(x, nattr0, we[0])[0]

    x0 = _layer([_gather(x0, src0)], [_gather(x0, dst0)], [x0],
                geom0, nattr0, dst0, N0, N0,
                p['Wm1_0'], p['Wm2_0'], p['Wu_0'], True, True, 1000)
    copy0 = x0

    cl0g = jnp.pad(cluster0, (0, MGN - N0))
    x1, pos1 = _pool(x0, pos0, cluster0, N1P, 512, N1)
    e1 = _compose(cluster0, eip.reshape(-1)).reshape(2, EP)
    e1s, e1d = e1[0], e1[1]
    geom1, nattr1 = _o3(pos1, e1s, e1d, N1P, 512)

    x1 = _layer([_gather(x1, e1s)], [_gather(x1, e1d)], [x1],
                geom1, nattr1, e1d, N1P, N1,
                p['Wm1_1'], p['Wm2_1'], p['Wu_1'], True, True, 512)
    copy1 = x1

    cl1p = jnp.pad(cluster1, (0, N1P - N1))
    cl1g = jnp.pad(cluster1, (0, MGN - N1))
    x2, pos2 = _pool(x1, pos1, cl1p, N2P, 640, N2)
    e2 = _compose(cluster1, e1.reshape(-1)).reshape(2, EP)
    e2s, e2d = e2[0], e2[1]
    geom2, nattr2 = _o3(pos2, e2s, e2d, N2P, 640)

    for i in (2, 3, 4):
        x2 = _layer([_gather(x2, e2s)], [_gather(x2, e2d)], [x2],
                    geom2, nattr2, e2d, N2P, N2,
                    p['Wm1_%d' % i], p['Wm2_%d' % i], p['Wu_%d' % i],
                    True, True, 640)

    x2up = _gather(x2, cl1g)[:N1P]
    x1 = _layer([_gather(x2, e2s), _gather(copy1, e1s)],
                [_gather(x2, e2d), _gather(copy1, e1d)],
                [x2up, copy1],
                geom1, nattr1, e1d, N1P, N1,
                p['Wm1_5'], p['Wm2_5'], p['Wu_5'], False, True, 512)

    x1up = _gather(x1, cl0g)[:N0]
    x0 = _layer([_gather(x1, e1s), _gather(copy0, src0)],
                [_gather(x1, e1d), _gather(copy0, dst0)],
                [x1up, copy0],
                geom0, nattr0, dst0, N0, N0,
                p['Wm1_6'], p['Wm2_6'], p['Wu_6'], False, True, 1000)

    x0 = _layer([_gather(x0, src0)], [_gather(x0, dst0)], [x0],
                geom0, nattr0, dst0, N0, N0,
                p['Wm1_7'], p['Wm2_7'], p['Wu_7'], True, False, 1000,
                final_ws=(p['W_amb1'], p['W_amb2']))
    return x0
